# baseline (speedup 1.0000x reference)
import jax
import jax.numpy as jnp
from jax import lax
from jax.experimental import pallas as pl
from jax.experimental.pallas import tpu as pltpu
from jax.experimental.pallas import tpu_sc as plsc

INPUT_SIZE = 1024
HIDDEN_SIZE = 4096
OUTPUT_SIZE = 1024
NUM_EXPERTS = 8
NUM_SELECTS = 2
H_PER_EXPERT = HIDDEN_SIZE // NUM_EXPERTS

_T = 2 * 2048
_P = _T * NUM_SELECTS
_BT = 256
_NB = _P // _BT + NUM_EXPERTS
_P_PAD = _NB * _BT

_GATE_BT = 512
_CMB_BT = 512

_NW = 32
_PPW = _P // _NW
_GCH = 64
_NCH = _PPW // _GCH
_L = 16
_DW = INPUT_SIZE // 2
_OW = OUTPUT_SIZE // 2


def _gate_kernel(x_ref, wg1_ref, wg2_ref, x16_ref, eidx_ref, esc_ref,
                 rank_ref, imp_ref, cnt_ref, loss_ref, pad_ref, be_ref):
    i = pl.program_id(0)
    nb = pl.num_programs(0)
    xb = x_ref[...]
    h = jnp.tanh(lax.dot_general(xb, wg1_ref[...], (((1,), (1,)), ((), ())),
                                 preferred_element_type=jnp.float32))
    logits = lax.dot_general(h, wg2_ref[...], (((1,), (1,)), ((), ())),
                             preferred_element_type=jnp.float32)
    lane = lax.broadcasted_iota(jnp.int32, logits.shape, 1)
    m1 = jnp.max(logits, axis=1, keepdims=True)
    i1 = jnp.min(jnp.where(logits == m1, lane, NUM_EXPERTS), axis=1,
                 keepdims=True)
    masked = jnp.where(lane == i1, -jnp.inf, logits)
    m2 = jnp.max(masked, axis=1, keepdims=True)
    i2 = jnp.min(jnp.where(masked == m2, lane, NUM_EXPERTS), axis=1,
                 keepdims=True)
    e2 = jnp.exp(m2 - m1)
    denom = 1.0 + e2
    s1 = 1.0 / denom
    s2 = e2 / denom
    sel1 = lane == i1
    sel2 = lane == i2

    x16_ref[...] = xb.astype(jnp.bfloat16)
    eidx_ref[...] = jnp.concatenate([i1, i2], axis=1)
    esc_ref[...] = jnp.concatenate([s1, s2], axis=1)

    @pl.when(i == 0)
    def _():
        imp_ref[...] = jnp.zeros_like(imp_ref)
        cnt_ref[...] = jnp.zeros_like(cnt_ref)

    base = cnt_ref[0:1, :]
    cnt_te = sel1.astype(jnp.int32) + sel2.astype(jnp.int32)
    n = cnt_te.shape[0]
    tri = (lax.broadcasted_iota(jnp.int32, (n, n), 0)
           > lax.broadcasted_iota(jnp.int32, (n, n), 1)).astype(jnp.float32)
    prev_f = lax.dot_general(tri, cnt_te.astype(jnp.float32),
                             (((1,), (0,)), ((), ())),
                             preferred_element_type=jnp.float32)
    prev = prev_f.astype(jnp.int32) + base
    r1 = jnp.sum(jnp.where(sel1, prev, 0), axis=1, keepdims=True)
    r2 = jnp.sum(jnp.where(sel2, prev, 0), axis=1, keepdims=True)
    rank_ref[...] = jnp.concatenate([r1, r2], axis=1)

    imp_part = jnp.sum(jnp.where(sel1, s1, 0.0) + jnp.where(sel2, s2, 0.0),
                       axis=0, keepdims=True)
    imp_ref[0:1, :] += imp_part
    cnt_ref[0:1, :] += jnp.sum(cnt_te, axis=0, keepdims=True)

    @pl.when(i == nb - 1)
    def _():
        imp = imp_ref[0:1, :]
        cnt = cnt_ref[0:1, :].astype(jnp.float32)

        def cv2(v):
            mean = jnp.sum(v) / NUM_EXPERTS
            var = jnp.sum((v - mean) ** 2) / (NUM_EXPERTS - 1)
            return var / (mean * mean + 1e-10)

        loss_ref[...] = jnp.full_like(loss_ref, 0.01 * (cv2(imp) + cv2(cnt)))

        padded = ((cnt_ref[0:1, :] + (_BT - 1)) // _BT) * _BT
        pad_ref[...] = jnp.zeros_like(pad_ref)
        pad_ref[0:1, 0:NUM_EXPERTS] = padded
        padf = padded.astype(jnp.float32)
        tri8 = (lax.broadcasted_iota(jnp.int32, (NUM_EXPERTS, NUM_EXPERTS), 0)
                > lax.broadcasted_iota(jnp.int32, (NUM_EXPERTS, NUM_EXPERTS),
                                       1)).astype(jnp.float32)
        offs_col = jnp.sum(tri8 * padf, axis=1, keepdims=True)
        bvals = (lax.broadcasted_iota(jnp.int32, (1, _NB), 1)
                 * _BT).astype(jnp.float32)
        be = jnp.sum((offs_col <= bvals).astype(jnp.int32), axis=0,
                     keepdims=True) - 1
        be_ref[0:1, :] = be


def _pos_kernel(eidx_ref, rank_ref, pad_ref, pos_ref):
    e = eidx_ref[...]
    lane8 = lax.broadcasted_iota(jnp.int32, (1, NUM_EXPERTS), 1)
    padf = pad_ref[0:1, 0:NUM_EXPERTS].astype(jnp.float32)
    acc = rank_ref[...]
    for ei in range(1, NUM_EXPERTS):
        off_i = jnp.sum(jnp.where(lane8 < ei, padf, 0.0)).astype(jnp.int32)
        acc = acc + jnp.where(e == ei, off_i, 0)
    pos_ref[...] = acc


def _ffn_kernel(be_ref, xs_ref, wg_ref, wu_ref, wd_ref, out_ref):
    del be_ref
    xb = xs_ref[...]
    hg = lax.dot_general(xb, wg_ref[0], (((1,), (1,)), ((), ())),
                         preferred_element_type=jnp.float32)
    hu = lax.dot_general(xb, wu_ref[0], (((1,), (1,)), ((), ())),
                         preferred_element_type=jnp.float32)
    h = (hg * jax.nn.sigmoid(hg) * hu).astype(jnp.bfloat16)
    o = lax.dot_general(h, wd_ref[0], (((1,), (1,)), ((), ())),
                        preferred_element_type=jnp.float32)
    out_ref[...] = o.astype(jnp.bfloat16)


def _cmb_kernel(op_ref, esc_ref, y_ref):
    o = op_ref[...].astype(jnp.float32)
    s = esc_ref[...]
    y_ref[...] = s[:, 0:1] * o[:, 0, :] + s[:, 1:2] * o[:, 1, :]


def _sc_dispatch_body(x_hbm, tok_hbm, pos_hbm, xs_hbm,
                      tok_v, pos_v, rowa_v, rowb_v, sga, sgb, ssc):
    wid = lax.axis_index("s") * 2 + lax.axis_index("c")
    pltpu.sync_copy(tok_hbm.at[wid], tok_v)
    pltpu.sync_copy(pos_hbm.at[wid], pos_v)
    bufs = (rowa_v, rowb_v)
    gsems = (sga, sgb)
    cps = [pltpu.async_copy(x_hbm.at[tok_v.at[c]], bufs[c % 2], gsems[c % 2])
           for c in range(2)]
    for c in range(_NCH):
        cps[c % 2].wait()
        pltpu.async_copy(bufs[c % 2], xs_hbm.at[pos_v.at[c]], ssc).wait()
        if c + 2 < _NCH:
            cps[c % 2] = pltpu.async_copy(x_hbm.at[tok_v.at[c + 2]],
                                          bufs[c % 2], gsems[c % 2])


def _sc_collect_body(os_hbm, pos_hbm, op_hbm, pos_v, rowa_v, rowb_v,
                     sga, sgb):
    wid = lax.axis_index("s") * 2 + lax.axis_index("c")
    base = wid * _PPW
    pltpu.sync_copy(pos_hbm.at[wid], pos_v)
    bufs = (rowa_v, rowb_v)
    gsems = (sga, sgb)
    cps = [pltpu.async_copy(os_hbm.at[pos_v.at[c]], bufs[c % 2], gsems[c % 2])
           for c in range(2)]
    for c in range(_NCH):
        cps[c % 2].wait()
        pltpu.sync_copy(bufs[c % 2], op_hbm.at[pl.ds(base + c * _GCH, _GCH)])
        if c + 2 < _NCH:
            cps[c % 2] = pltpu.async_copy(os_hbm.at[pos_v.at[c + 2]],
                                          bufs[c % 2], gsems[c % 2])


_sc_mesh = plsc.VectorSubcoreMesh(core_axis_name="c", subcore_axis_name="s")

_sc_dispatch = pl.kernel(
    _sc_dispatch_body, mesh=_sc_mesh,
    out_type=jax.ShapeDtypeStruct((_P_PAD, _DW), jnp.int32),
    scratch_types=[
        pltpu.VMEM((_NCH, _GCH), jnp.int32),
        pltpu.VMEM((_NCH, _GCH), jnp.int32),
        pltpu.VMEM((_GCH, _DW), jnp.int32),
        pltpu.VMEM((_GCH, _DW), jnp.int32),
        pltpu.SemaphoreType.DMA,
        pltpu.SemaphoreType.DMA,
        pltpu.SemaphoreType.DMA,
    ],
)

_sc_collect = pl.kernel(
    _sc_collect_body, mesh=_sc_mesh,
    out_type=jax.ShapeDtypeStruct((_P, _OW), jnp.int32),
    scratch_types=[
        pltpu.VMEM((_NCH, _GCH), jnp.int32),
        pltpu.VMEM((_GCH, _OW), jnp.int32),
        pltpu.VMEM((_GCH, _OW), jnp.int32),
        pltpu.SemaphoreType.DMA,
        pltpu.SemaphoreType.DMA,
    ],
)


@jax.jit
def kernel(x, Wg1, Wg2, W_gate, W_up, W_down):
    B, S, D = x.shape
    xf = x.reshape(-1, D)
    T = xf.shape[0]
    E = NUM_EXPERTS

    nb_gate = T // _GATE_BT
    x16, eidx, esc, rank, imp, cnt, loss, pad, be = pl.pallas_call(
        _gate_kernel,
        grid=(nb_gate,),
        in_specs=[
            pl.BlockSpec((_GATE_BT, D), lambda i: (i, 0)),
            pl.BlockSpec((E, D), lambda i: (0, 0)),
            pl.BlockSpec((E, E), lambda i: (0, 0)),
        ],
        out_specs=[
            pl.BlockSpec((_GATE_BT, D), lambda i: (i, 0)),
            pl.BlockSpec((_GATE_BT, 2), lambda i: (i, 0)),
            pl.BlockSpec((_GATE_BT, 2), lambda i: (i, 0)),
            pl.BlockSpec((_GATE_BT, 2), lambda i: (i, 0)),
            pl.BlockSpec((8, E), lambda i: (0, 0)),
            pl.BlockSpec((8, E), lambda i: (0, 0)),
            pl.BlockSpec((8, E), lambda i: (0, 0)),
            pl.BlockSpec((8, _L), lambda i: (0, 0)),
            pl.BlockSpec((8, _NB), lambda i: (0, 0)),
        ],
        out_shape=[
            jax.ShapeDtypeStruct((T, D), jnp.bfloat16),
            jax.ShapeDtypeStruct((T, 2), jnp.int32),
            jax.ShapeDtypeStruct((T, 2), jnp.float32),
            jax.ShapeDtypeStruct((T, 2), jnp.int32),
            jax.ShapeDtypeStruct((8, E), jnp.float32),
            jax.ShapeDtypeStruct((8, E), jnp.int32),
            jax.ShapeDtypeStruct((8, E), jnp.float32),
            jax.ShapeDtypeStruct((8, _L), jnp.int32),
            jax.ShapeDtypeStruct((8, _NB), jnp.int32),
        ],
    )(xf, Wg1, Wg2)

    importance = imp[0]
    load = cnt[0]
    balance_loss = loss[0, 0]
    block_expert = be[0]

    pos = pl.pallas_call(
        _pos_kernel,
        grid=(1,),
        in_specs=[
            pl.BlockSpec((T, 2), lambda i: (0, 0)),
            pl.BlockSpec((T, 2), lambda i: (0, 0)),
            pl.BlockSpec((8, _L), lambda i: (0, 0)),
        ],
        out_specs=pl.BlockSpec((T, 2), lambda i: (0, 0)),
        out_shape=jax.ShapeDtypeStruct((T, 2), jnp.int32),
    )(eidx, rank, pad)

    tok_ids = (jnp.arange(_P, dtype=jnp.int32) // NUM_SELECTS).reshape(
        _NW, _NCH, _GCH)
    pos_w = pos.reshape(_NW, _NCH, _GCH)

    xi = lax.bitcast_convert_type(x16.reshape(T, _DW, 2), jnp.int32)
    xs_i = _sc_dispatch(xi, tok_ids, pos_w)
    xs = lax.bitcast_convert_type(xs_i, jnp.bfloat16).reshape(_P_PAD, D)

    wg16 = W_gate.astype(jnp.bfloat16)
    wu16 = W_up.astype(jnp.bfloat16)
    wd16 = W_down.astype(jnp.bfloat16)

    out_slots = pl.pallas_call(
        _ffn_kernel,
        grid_spec=pltpu.PrefetchScalarGridSpec(
            num_scalar_prefetch=1,
            grid=(_NB,),
            in_specs=[
                pl.BlockSpec((_BT, D), lambda b, be_: (b, 0)),
                pl.BlockSpec((1, H_PER_EXPERT, D),
                             lambda b, be_: (be_[b], 0, 0)),
                pl.BlockSpec((1, H_PER_EXPERT, D),
                             lambda b, be_: (be_[b], 0, 0)),
                pl.BlockSpec((1, OUTPUT_SIZE, H_PER_EXPERT),
                             lambda b, be_: (be_[b], 0, 0)),
            ],
            out_specs=pl.BlockSpec((_BT, OUTPUT_SIZE), lambda b, be_: (b, 0)),
        ),
        out_shape=jax.ShapeDtypeStruct((_P_PAD, OUTPUT_SIZE), jnp.bfloat16),
    )(block_expert, xs, wg16, wu16, wd16)

    os_i = lax.bitcast_convert_type(
        out_slots.reshape(_P_PAD, _OW, 2), jnp.int32)
    op_i = _sc_collect(os_i, pos_w)
    out_pairs = lax.bitcast_convert_type(op_i, jnp.bfloat16).reshape(
        T, 2, OUTPUT_SIZE)

    y = pl.pallas_call(
        _cmb_kernel,
        grid=(T // _CMB_BT,),
        in_specs=[
            pl.BlockSpec((_CMB_BT, 2, OUTPUT_SIZE), lambda i: (i, 0, 0)),
            pl.BlockSpec((_CMB_BT, 2), lambda i: (i, 0)),
        ],
        out_specs=pl.BlockSpec((_CMB_BT, OUTPUT_SIZE), lambda i: (i, 0)),
        out_shape=jax.ShapeDtypeStruct((T, OUTPUT_SIZE), jnp.float32),
    )(out_pairs, esc)

    return (y.reshape(B, S, OUTPUT_SIZE), balance_loss, load, importance)


# speedup vs baseline: 13.1580x; 13.1580x over previous
import jax
import jax.numpy as jnp
from jax import lax
from jax.experimental import pallas as pl
from jax.experimental.pallas import tpu as pltpu
from jax.experimental.pallas import tpu_sc as plsc

INPUT_SIZE = 1024
HIDDEN_SIZE = 4096
OUTPUT_SIZE = 1024
NUM_EXPERTS = 8
NUM_SELECTS = 2
H_PER_EXPERT = HIDDEN_SIZE // NUM_EXPERTS

_T = 2 * 2048
_P = _T * NUM_SELECTS
_BT = 256
_NB = _P // _BT + NUM_EXPERTS
_P_PAD = _NB * _BT

_GATE_BT = 512
_CMB_BT = 512

_NW = 32
_PPW = _P // _NW
_GCH = 64
_NCH = _PPW // _GCH
_L = 16
_DW = INPUT_SIZE // 2
_OW = OUTPUT_SIZE // 2


def _pack_bf16(a):
    u = lax.bitcast_convert_type(a, jnp.uint32)
    b = (u + jnp.uint32(0x7FFF) + ((u >> 16) & jnp.uint32(1))) >> 16
    n = a.shape[-1] // 2
    lo = b[..., :n]
    hi = b[..., n:]
    return lax.bitcast_convert_type(lo | (hi << 16), jnp.int32)


def _unpack_f32(p):
    u = lax.bitcast_convert_type(p, jnp.uint32)
    lo = lax.bitcast_convert_type(u << 16, jnp.float32)
    hi = lax.bitcast_convert_type(u & jnp.uint32(0xFFFF0000), jnp.float32)
    return jnp.concatenate([lo, hi], axis=-1)


def _gate_kernel(x_ref, wg1_ref, wg2_ref, x16i_ref, eidx_ref, esc_ref,
                 rank_ref, imp_ref, cnt_ref, loss_ref, pad_ref, be_ref):
    i = pl.program_id(0)
    nb = pl.num_programs(0)
    xb = x_ref[...]
    h = jnp.tanh(lax.dot_general(xb, wg1_ref[...], (((1,), (1,)), ((), ())),
                                 preferred_element_type=jnp.float32))
    logits = lax.dot_general(h, wg2_ref[...], (((1,), (1,)), ((), ())),
                             preferred_element_type=jnp.float32)
    lane = lax.broadcasted_iota(jnp.int32, logits.shape, 1)
    m1 = jnp.max(logits, axis=1, keepdims=True)
    i1 = jnp.min(jnp.where(logits == m1, lane, NUM_EXPERTS), axis=1,
                 keepdims=True)
    masked = jnp.where(lane == i1, -jnp.inf, logits)
    m2 = jnp.max(masked, axis=1, keepdims=True)
    i2 = jnp.min(jnp.where(masked == m2, lane, NUM_EXPERTS), axis=1,
                 keepdims=True)
    e2 = jnp.exp(m2 - m1)
    denom = 1.0 + e2
    s1 = 1.0 / denom
    s2 = e2 / denom
    sel1 = lane == i1
    sel2 = lane == i2

    x16i_ref[...] = _pack_bf16(xb)
    eidx_ref[...] = jnp.concatenate([i1, i2], axis=1)
    esc_ref[...] = jnp.concatenate([s1, s2], axis=1)

    @pl.when(i == 0)
    def _():
        imp_ref[...] = jnp.zeros_like(imp_ref)
        cnt_ref[...] = jnp.zeros_like(cnt_ref)

    base = cnt_ref[0:1, :]
    cnt_te = sel1.astype(jnp.int32) + sel2.astype(jnp.int32)
    n = cnt_te.shape[0]
    tri = (lax.broadcasted_iota(jnp.int32, (n, n), 0)
           > lax.broadcasted_iota(jnp.int32, (n, n), 1)).astype(jnp.float32)
    prev_f = lax.dot_general(tri, cnt_te.astype(jnp.float32),
                             (((1,), (0,)), ((), ())),
                             preferred_element_type=jnp.float32)
    prev = prev_f.astype(jnp.int32) + base
    r1 = jnp.sum(jnp.where(sel1, prev, 0), axis=1, keepdims=True)
    r2 = jnp.sum(jnp.where(sel2, prev, 0), axis=1, keepdims=True)
    rank_ref[...] = jnp.concatenate([r1, r2], axis=1)

    imp_part = jnp.sum(jnp.where(sel1, s1, 0.0) + jnp.where(sel2, s2, 0.0),
                       axis=0, keepdims=True)
    imp_ref[0:1, :] += imp_part
    cnt_ref[0:1, :] += jnp.sum(cnt_te, axis=0, keepdims=True)

    @pl.when(i == nb - 1)
    def _():
        imp = imp_ref[0:1, :]
        cnt = cnt_ref[0:1, :].astype(jnp.float32)

        def cv2(v):
            mean = jnp.sum(v) / NUM_EXPERTS
            var = jnp.sum((v - mean) ** 2) / (NUM_EXPERTS - 1)
            return var / (mean * mean + 1e-10)

        loss_ref[...] = jnp.full_like(loss_ref, 0.01 * (cv2(imp) + cv2(cnt)))

        padded = ((cnt_ref[0:1, :] + (_BT - 1)) // _BT) * _BT
        pad_ref[...] = jnp.zeros_like(pad_ref)
        pad_ref[0:1, 0:NUM_EXPERTS] = padded
        padf = padded.astype(jnp.float32)
        tri8 = (lax.broadcasted_iota(jnp.int32, (NUM_EXPERTS, NUM_EXPERTS), 0)
                > lax.broadcasted_iota(jnp.int32, (NUM_EXPERTS, NUM_EXPERTS),
                                       1)).astype(jnp.float32)
        offs_col = jnp.sum(tri8 * padf, axis=1, keepdims=True)
        bvals = (lax.broadcasted_iota(jnp.int32, (1, _NB), 1)
                 * _BT).astype(jnp.float32)
        be = jnp.sum((offs_col <= bvals).astype(jnp.int32), axis=0,
                     keepdims=True) - 1
        be_ref[0:1, :] = be


def _pos_kernel(eidx_ref, rank_ref, pad_ref, pos_ref):
    e = eidx_ref[...]
    lane8 = lax.broadcasted_iota(jnp.int32, (1, NUM_EXPERTS), 1)
    padf = pad_ref[0:1, 0:NUM_EXPERTS].astype(jnp.float32)
    acc = rank_ref[...]
    for ei in range(1, NUM_EXPERTS):
        off_i = jnp.sum(jnp.where(lane8 < ei, padf, 0.0)).astype(jnp.int32)
        acc = acc + jnp.where(e == ei, off_i, 0)
    pos_ref[...] = acc


def _ffn_kernel(be_ref, xs_ref, wg_ref, wu_ref, wd_ref, out_ref):
    del be_ref
    xb = _unpack_f32(xs_ref[...]).astype(jnp.bfloat16)
    hg = lax.dot_general(xb, wg_ref[0], (((1,), (1,)), ((), ())),
                         preferred_element_type=jnp.float32)
    hu = lax.dot_general(xb, wu_ref[0], (((1,), (1,)), ((), ())),
                         preferred_element_type=jnp.float32)
    h = (hg * jax.nn.sigmoid(hg) * hu).astype(jnp.bfloat16)
    o = lax.dot_general(h, wd_ref[0], (((1,), (1,)), ((), ())),
                        preferred_element_type=jnp.float32)
    out_ref[...] = _pack_bf16(o)


def _cmb_kernel(op_ref, esc_ref, y_ref):
    o = _unpack_f32(op_ref[...])
    s = esc_ref[...]
    y_ref[...] = s[:, 0:1] * o[:, 0, :] + s[:, 1:2] * o[:, 1, :]


def _sc_dispatch_body(x_hbm, tok_hbm, pos_hbm, xs_hbm,
                      tok_v, pos_v, rowa_v, rowb_v, rowc_v,
                      sga, sgb, sgc, ssc):
    wid = lax.axis_index("s") * 2 + lax.axis_index("c")
    pltpu.sync_copy(tok_hbm.at[wid], tok_v)
    pltpu.sync_copy(pos_hbm.at[wid], pos_v)
    bufs = (rowa_v, rowb_v, rowc_v)
    gsems = (sga, sgb, sgc)
    cps = [pltpu.async_copy(x_hbm.at[tok_v.at[c]], bufs[c % 3], gsems[c % 3])
           for c in range(min(3, _NCH))]
    for c in range(_NCH):
        cps[c % 3].wait()
        pltpu.async_copy(bufs[c % 3], xs_hbm.at[pos_v.at[c]], ssc).wait()
        if c + 3 < _NCH:
            cps[c % 3] = pltpu.async_copy(x_hbm.at[tok_v.at[c + 3]],
                                          bufs[c % 3], gsems[c % 3])


def _sc_collect_body(os_hbm, pos_hbm, op_hbm, pos_v, rowa_v, rowb_v, rowc_v,
                     sga, sgb, sgc):
    wid = lax.axis_index("s") * 2 + lax.axis_index("c")
    base = wid * _PPW
    pltpu.sync_copy(pos_hbm.at[wid], pos_v)
    bufs = (rowa_v, rowb_v, rowc_v)
    gsems = (sga, sgb, sgc)
    cps = [pltpu.async_copy(os_hbm.at[pos_v.at[c]], bufs[c % 3], gsems[c % 3])
           for c in range(min(3, _NCH))]
    for c in range(_NCH):
        cps[c % 3].wait()
        pltpu.sync_copy(bufs[c % 3], op_hbm.at[pl.ds(base + c * _GCH, _GCH)])
        if c + 3 < _NCH:
            cps[c % 3] = pltpu.async_copy(os_hbm.at[pos_v.at[c + 3]],
                                          bufs[c % 3], gsems[c % 3])


_sc_mesh = plsc.VectorSubcoreMesh(core_axis_name="c", subcore_axis_name="s")

_sc_dispatch = pl.kernel(
    _sc_dispatch_body, mesh=_sc_mesh,
    out_type=jax.ShapeDtypeStruct((_P_PAD, _DW), jnp.int32),
    scratch_types=[
        pltpu.VMEM((_NCH, _GCH), jnp.int32),
        pltpu.VMEM((_NCH, _GCH), jnp.int32),
        pltpu.VMEM((_GCH, _DW), jnp.int32),
        pltpu.VMEM((_GCH, _DW), jnp.int32),
        pltpu.VMEM((_GCH, _DW), jnp.int32),
        pltpu.SemaphoreType.DMA,
        pltpu.SemaphoreType.DMA,
        pltpu.SemaphoreType.DMA,
        pltpu.SemaphoreType.DMA,
    ],
)

_sc_collect = pl.kernel(
    _sc_collect_body, mesh=_sc_mesh,
    out_type=jax.ShapeDtypeStruct((_P, _OW), jnp.int32),
    scratch_types=[
        pltpu.VMEM((_NCH, _GCH), jnp.int32),
        pltpu.VMEM((_GCH, _OW), jnp.int32),
        pltpu.VMEM((_GCH, _OW), jnp.int32),
        pltpu.VMEM((_GCH, _OW), jnp.int32),
        pltpu.SemaphoreType.DMA,
        pltpu.SemaphoreType.DMA,
        pltpu.SemaphoreType.DMA,
    ],
)


@jax.jit
def kernel(x, Wg1, Wg2, W_gate, W_up, W_down):
    B, S, D = x.shape
    xf = x.reshape(-1, D)
    T = xf.shape[0]
    E = NUM_EXPERTS

    nb_gate = T // _GATE_BT
    x16i, eidx, esc, rank, imp, cnt, loss, pad, be = pl.pallas_call(
        _gate_kernel,
        grid=(nb_gate,),
        in_specs=[
            pl.BlockSpec((_GATE_BT, D), lambda i: (i, 0)),
            pl.BlockSpec((E, D), lambda i: (0, 0)),
            pl.BlockSpec((E, E), lambda i: (0, 0)),
        ],
        out_specs=[
            pl.BlockSpec((_GATE_BT, _DW), lambda i: (i, 0)),
            pl.BlockSpec((_GATE_BT, 2), lambda i: (i, 0)),
            pl.BlockSpec((_GATE_BT, 2), lambda i: (i, 0)),
            pl.BlockSpec((_GATE_BT, 2), lambda i: (i, 0)),
            pl.BlockSpec((8, E), lambda i: (0, 0)),
            pl.BlockSpec((8, E), lambda i: (0, 0)),
            pl.BlockSpec((8, E), lambda i: (0, 0)),
            pl.BlockSpec((8, _L), lambda i: (0, 0)),
            pl.BlockSpec((8, _NB), lambda i: (0, 0)),
        ],
        out_shape=[
            jax.ShapeDtypeStruct((T, _DW), jnp.int32),
            jax.ShapeDtypeStruct((T, 2), jnp.int32),
            jax.ShapeDtypeStruct((T, 2), jnp.float32),
            jax.ShapeDtypeStruct((T, 2), jnp.int32),
            jax.ShapeDtypeStruct((8, E), jnp.float32),
            jax.ShapeDtypeStruct((8, E), jnp.int32),
            jax.ShapeDtypeStruct((8, E), jnp.float32),
            jax.ShapeDtypeStruct((8, _L), jnp.int32),
            jax.ShapeDtypeStruct((8, _NB), jnp.int32),
        ],
    )(xf, Wg1, Wg2)

    importance = imp[0]
    load = cnt[0]
    balance_loss = loss[0, 0]
    block_expert = be[0]

    pos = pl.pallas_call(
        _pos_kernel,
        grid=(1,),
        in_specs=[
            pl.BlockSpec((T, 2), lambda i: (0, 0)),
            pl.BlockSpec((T, 2), lambda i: (0, 0)),
            pl.BlockSpec((8, _L), lambda i: (0, 0)),
        ],
        out_specs=pl.BlockSpec((T, 2), lambda i: (0, 0)),
        out_shape=jax.ShapeDtypeStruct((T, 2), jnp.int32),
    )(eidx, rank, pad)

    tok_ids = (jnp.arange(_P, dtype=jnp.int32) // NUM_SELECTS).reshape(
        _NW, _NCH, _GCH)
    pos_w = pos.reshape(_NW, _NCH, _GCH)

    xs = _sc_dispatch(x16i, tok_ids, pos_w)

    wg16 = W_gate.astype(jnp.bfloat16)
    wu16 = W_up.astype(jnp.bfloat16)
    wd16 = W_down.astype(jnp.bfloat16)

    out_slots = pl.pallas_call(
        _ffn_kernel,
        grid_spec=pltpu.PrefetchScalarGridSpec(
            num_scalar_prefetch=1,
            grid=(_NB,),
            in_specs=[
                pl.BlockSpec((_BT, _DW), lambda b, be_: (b, 0)),
                pl.BlockSpec((1, H_PER_EXPERT, D),
                             lambda b, be_: (be_[b], 0, 0)),
                pl.BlockSpec((1, H_PER_EXPERT, D),
                             lambda b, be_: (be_[b], 0, 0)),
                pl.BlockSpec((1, OUTPUT_SIZE, H_PER_EXPERT),
                             lambda b, be_: (be_[b], 0, 0)),
            ],
            out_specs=pl.BlockSpec((_BT, _OW), lambda b, be_: (b, 0)),
        ),
        out_shape=jax.ShapeDtypeStruct((_P_PAD, _OW), jnp.int32),
    )(block_expert, xs, wg16, wu16, wd16)

    out_pairs = _sc_collect(out_slots, pos_w)

    y = pl.pallas_call(
        _cmb_kernel,
        grid=(T // _CMB_BT,),
        in_specs=[
            pl.BlockSpec((_CMB_BT, 2, _OW), lambda i: (i, 0, 0)),
            pl.BlockSpec((_CMB_BT, 2), lambda i: (i, 0)),
        ],
        out_specs=pl.BlockSpec((_CMB_BT, OUTPUT_SIZE), lambda i: (i, 0)),
        out_shape=jax.ShapeDtypeStruct((T, OUTPUT_SIZE), jnp.float32),
    )(out_pairs.reshape(T, 2, _OW), esc)

    return (y.reshape(B, S, OUTPUT_SIZE), balance_loss, load, importance)


# speedup vs baseline: 13.8082x; 1.0494x over previous
import jax
import jax.numpy as jnp
from jax import lax
from jax.experimental import pallas as pl
from jax.experimental.pallas import tpu as pltpu
from jax.experimental.pallas import tpu_sc as plsc

INPUT_SIZE = 1024
HIDDEN_SIZE = 4096
OUTPUT_SIZE = 1024
NUM_EXPERTS = 8
NUM_SELECTS = 2
H_PER_EXPERT = HIDDEN_SIZE // NUM_EXPERTS

_T = 2 * 2048
_P = _T * NUM_SELECTS
_BT = 256
_NB = _P // _BT + NUM_EXPERTS
_VBPE = _T // _BT
_NVB = NUM_EXPERTS * _VBPE
_V_ROWS = (_NVB + 1) * _BT

_GATE_BT = 512
_CMB_BT = 512

_NW = 32
_PPW = _P // _NW
_GCH = 64
_NCH = _PPW // _GCH
_L = 16
_DW = INPUT_SIZE // 2
_OW = OUTPUT_SIZE // 2


def _pack_bf16(a):
    u = lax.bitcast_convert_type(a, jnp.uint32)
    b = (u + jnp.uint32(0x7FFF) + ((u >> 16) & jnp.uint32(1))) >> 16
    n = a.shape[-1] // 2
    lo = b[..., :n]
    hi = b[..., n:]
    return lax.bitcast_convert_type(lo | (hi << 16), jnp.int32)


def _unpack_f32(p):
    u = lax.bitcast_convert_type(p, jnp.uint32)
    lo = lax.bitcast_convert_type(u << 16, jnp.float32)
    hi = lax.bitcast_convert_type(u & jnp.uint32(0xFFFF0000), jnp.float32)
    return jnp.concatenate([lo, hi], axis=-1)


def _gate_kernel(x_ref, wg1_ref, wg2_ref, wge_ref, wue_ref, wde_ref,
                 x16i_ref, eidx_ref, esc_ref, vpos_ref,
                 imp_ref, cnt_ref, loss_ref, be_ref, vb_ref,
                 wg16_ref, wu16_ref, wd16_ref):
    i = pl.program_id(0)
    nb = pl.num_programs(0)
    xb = x_ref[...]
    h = jnp.tanh(lax.dot_general(xb, wg1_ref[...], (((1,), (1,)), ((), ())),
                                 preferred_element_type=jnp.float32))
    logits = lax.dot_general(h, wg2_ref[...], (((1,), (1,)), ((), ())),
                             preferred_element_type=jnp.float32)
    lane = lax.broadcasted_iota(jnp.int32, logits.shape, 1)
    m1 = jnp.max(logits, axis=1, keepdims=True)
    i1 = jnp.min(jnp.where(logits == m1, lane, NUM_EXPERTS), axis=1,
                 keepdims=True)
    masked = jnp.where(lane == i1, -jnp.inf, logits)
    m2 = jnp.max(masked, axis=1, keepdims=True)
    i2 = jnp.min(jnp.where(masked == m2, lane, NUM_EXPERTS), axis=1,
                 keepdims=True)
    e2 = jnp.exp(m2 - m1)
    denom = 1.0 + e2
    s1 = 1.0 / denom
    s2 = e2 / denom
    sel1 = lane == i1
    sel2 = lane == i2

    x16i_ref[...] = _pack_bf16(xb)
    eidx_ref[...] = jnp.concatenate([i1, i2], axis=1)
    esc_ref[...] = jnp.concatenate([s1, s2], axis=1)

    wg16_ref[...] = wge_ref[...].astype(jnp.bfloat16)
    wu16_ref[...] = wue_ref[...].astype(jnp.bfloat16)
    wd16_ref[...] = wde_ref[...].astype(jnp.bfloat16)

    @pl.when(i == 0)
    def _():
        imp_ref[...] = jnp.zeros_like(imp_ref)
        cnt_ref[...] = jnp.zeros_like(cnt_ref)

    base = cnt_ref[0:1, :]
    cnt_te = sel1.astype(jnp.int32) + sel2.astype(jnp.int32)
    n = cnt_te.shape[0]
    tri = (lax.broadcasted_iota(jnp.int32, (n, n), 0)
           > lax.broadcasted_iota(jnp.int32, (n, n), 1)).astype(jnp.float32)
    prev_f = lax.dot_general(tri, cnt_te.astype(jnp.float32),
                             (((1,), (0,)), ((), ())),
                             preferred_element_type=jnp.float32)
    prev = prev_f.astype(jnp.int32) + base
    r1 = jnp.sum(jnp.where(sel1, prev, 0), axis=1, keepdims=True)
    r2 = jnp.sum(jnp.where(sel2, prev, 0), axis=1, keepdims=True)
    vpos_ref[...] = (jnp.concatenate([i1, i2], axis=1) * _T
                     + jnp.concatenate([r1, r2], axis=1))

    imp_part = jnp.sum(jnp.where(sel1, s1, 0.0) + jnp.where(sel2, s2, 0.0),
                       axis=0, keepdims=True)
    imp_ref[0:1, :] += imp_part
    cnt_ref[0:1, :] += jnp.sum(cnt_te, axis=0, keepdims=True)

    @pl.when(i == nb - 1)
    def _():
        imp = imp_ref[0:1, :]
        cnt = cnt_ref[0:1, :].astype(jnp.float32)

        def cv2(v):
            mean = jnp.sum(v) / NUM_EXPERTS
            var = jnp.sum((v - mean) ** 2) / (NUM_EXPERTS - 1)
            return var / (mean * mean + 1e-10)

        loss_ref[...] = jnp.full_like(loss_ref, 0.01 * (cv2(imp) + cv2(cnt)))

        padded = ((cnt_ref[0:1, :] + (_BT - 1)) // _BT) * _BT
        padf = padded.astype(jnp.float32)
        tri8 = (lax.broadcasted_iota(jnp.int32, (NUM_EXPERTS, NUM_EXPERTS), 0)
                > lax.broadcasted_iota(jnp.int32, (NUM_EXPERTS, NUM_EXPERTS),
                                       1)).astype(jnp.float32)
        offs_col = jnp.sum(tri8 * padf, axis=1, keepdims=True)
        total = jnp.sum(padf)
        bvals = (lax.broadcasted_iota(jnp.int32, (1, _NB), 1)
                 * _BT).astype(jnp.float32)
        be = jnp.sum((offs_col <= bvals).astype(jnp.int32), axis=0,
                     keepdims=True) - 1
        lane_e = lax.broadcasted_iota(jnp.int32, (NUM_EXPERTS, _NB), 0)
        offs_be = jnp.sum(
            jnp.where(lane_e == be, offs_col, 0.0), axis=0,
            keepdims=True)
        inblk = ((bvals - offs_be) / _BT).astype(jnp.int32)
        vb = jnp.where(bvals < total, be * _VBPE + inblk, _NVB)
        be_ref[0:1, :] = be
        vb_ref[0:1, :] = vb


def _ffn_kernel(be_ref, vb_ref, xs_ref, wg_ref, wu_ref, wd_ref, out_ref):
    del be_ref, vb_ref
    xb = _unpack_f32(xs_ref[...]).astype(jnp.bfloat16)
    hg = lax.dot_general(xb, wg_ref[0], (((1,), (1,)), ((), ())),
                         preferred_element_type=jnp.float32)
    hu = lax.dot_general(xb, wu_ref[0], (((1,), (1,)), ((), ())),
                         preferred_element_type=jnp.float32)
    h = (hg * jax.nn.sigmoid(hg) * hu).astype(jnp.bfloat16)
    o = lax.dot_general(h, wd_ref[0], (((1,), (1,)), ((), ())),
                        preferred_element_type=jnp.float32)
    out_ref[...] = _pack_bf16(o)


def _cmb_kernel(op_ref, esc_ref, y_ref):
    o = _unpack_f32(op_ref[...])
    s = esc_ref[...]
    y_ref[...] = s[:, 0:1] * o[:, 0, :] + s[:, 1:2] * o[:, 1, :]


def _sc_dispatch_body(x_hbm, tok_hbm, pos_hbm, xs_hbm,
                      tok_v, pos_v, rowa_v, rowb_v, rowc_v,
                      sga, sgb, sgc, ssc):
    wid = lax.axis_index("s") * 2 + lax.axis_index("c")
    pltpu.sync_copy(tok_hbm.at[wid], tok_v)
    pltpu.sync_copy(pos_hbm.at[wid], pos_v)
    bufs = (rowa_v, rowb_v, rowc_v)
    gsems = (sga, sgb, sgc)
    cps = [pltpu.async_copy(x_hbm.at[tok_v.at[c]], bufs[c % 3], gsems[c % 3])
           for c in range(min(3, _NCH))]
    for c in range(_NCH):
        cps[c % 3].wait()
        pltpu.async_copy(bufs[c % 3], xs_hbm.at[pos_v.at[c]], ssc).wait()
        if c + 3 < _NCH:
            cps[c % 3] = pltpu.async_copy(x_hbm.at[tok_v.at[c + 3]],
                                          bufs[c % 3], gsems[c % 3])


def _sc_collect_body(os_hbm, pos_hbm, op_hbm, pos_v, rowa_v, rowb_v, rowc_v,
                     sga, sgb, sgc):
    wid = lax.axis_index("s") * 2 + lax.axis_index("c")
    base = wid * _PPW
    pltpu.sync_copy(pos_hbm.at[wid], pos_v)
    bufs = (rowa_v, rowb_v, rowc_v)
    gsems = (sga, sgb, sgc)
    cps = [pltpu.async_copy(os_hbm.at[pos_v.at[c]], bufs[c % 3], gsems[c % 3])
           for c in range(min(3, _NCH))]
    for c in range(_NCH):
        cps[c % 3].wait()
        pltpu.sync_copy(bufs[c % 3], op_hbm.at[pl.ds(base + c * _GCH, _GCH)])
        if c + 3 < _NCH:
            cps[c % 3] = pltpu.async_copy(os_hbm.at[pos_v.at[c + 3]],
                                          bufs[c % 3], gsems[c % 3])


_sc_mesh = plsc.VectorSubcoreMesh(core_axis_name="c", subcore_axis_name="s")

_sc_dispatch = pl.kernel(
    _sc_dispatch_body, mesh=_sc_mesh,
    out_type=jax.ShapeDtypeStruct((_V_ROWS, _DW), jnp.int32),
    scratch_types=[
        pltpu.VMEM((_NCH, _GCH), jnp.int32),
        pltpu.VMEM((_NCH, _GCH), jnp.int32),
        pltpu.VMEM((_GCH, _DW), jnp.int32),
        pltpu.VMEM((_GCH, _DW), jnp.int32),
        pltpu.VMEM((_GCH, _DW), jnp.int32),
        pltpu.SemaphoreType.DMA,
        pltpu.SemaphoreType.DMA,
        pltpu.SemaphoreType.DMA,
        pltpu.SemaphoreType.DMA,
    ],
)

_sc_collect = pl.kernel(
    _sc_collect_body, mesh=_sc_mesh,
    out_type=jax.ShapeDtypeStruct((_P, _OW), jnp.int32),
    scratch_types=[
        pltpu.VMEM((_NCH, _GCH), jnp.int32),
        pltpu.VMEM((_GCH, _OW), jnp.int32),
        pltpu.VMEM((_GCH, _OW), jnp.int32),
        pltpu.VMEM((_GCH, _OW), jnp.int32),
        pltpu.SemaphoreType.DMA,
        pltpu.SemaphoreType.DMA,
        pltpu.SemaphoreType.DMA,
    ],
)


@jax.jit
def kernel(x, Wg1, Wg2, W_gate, W_up, W_down):
    B, S, D = x.shape
    xf = x.reshape(-1, D)
    T = xf.shape[0]
    E = NUM_EXPERTS

    nb_gate = T // _GATE_BT
    (x16i, eidx, esc, vpos, imp, cnt, loss, be, vb,
     wg16, wu16, wd16) = pl.pallas_call(
        _gate_kernel,
        grid=(nb_gate,),
        in_specs=[
            pl.BlockSpec((_GATE_BT, D), lambda i: (i, 0)),
            pl.BlockSpec((E, D), lambda i: (0, 0)),
            pl.BlockSpec((E, E), lambda i: (0, 0)),
            pl.BlockSpec((1, H_PER_EXPERT, D), lambda i: (i, 0, 0)),
            pl.BlockSpec((1, H_PER_EXPERT, D), lambda i: (i, 0, 0)),
            pl.BlockSpec((1, OUTPUT_SIZE, H_PER_EXPERT), lambda i: (i, 0, 0)),
        ],
        out_specs=[
            pl.BlockSpec((_GATE_BT, _DW), lambda i: (i, 0)),
            pl.BlockSpec((_GATE_BT, 2), lambda i: (i, 0)),
            pl.BlockSpec((_GATE_BT, 2), lambda i: (i, 0)),
            pl.BlockSpec((_GATE_BT, 2), lambda i: (i, 0)),
            pl.BlockSpec((8, E), lambda i: (0, 0)),
            pl.BlockSpec((8, E), lambda i: (0, 0)),
            pl.BlockSpec((8, E), lambda i: (0, 0)),
            pl.BlockSpec((8, _NB), lambda i: (0, 0)),
            pl.BlockSpec((8, _NB), lambda i: (0, 0)),
            pl.BlockSpec((1, H_PER_EXPERT, D), lambda i: (i, 0, 0)),
            pl.BlockSpec((1, H_PER_EXPERT, D), lambda i: (i, 0, 0)),
            pl.BlockSpec((1, OUTPUT_SIZE, H_PER_EXPERT), lambda i: (i, 0, 0)),
        ],
        out_shape=[
            jax.ShapeDtypeStruct((T, _DW), jnp.int32),
            jax.ShapeDtypeStruct((T, 2), jnp.int32),
            jax.ShapeDtypeStruct((T, 2), jnp.float32),
            jax.ShapeDtypeStruct((T, 2), jnp.int32),
            jax.ShapeDtypeStruct((8, E), jnp.float32),
            jax.ShapeDtypeStruct((8, E), jnp.int32),
            jax.ShapeDtypeStruct((8, E), jnp.float32),
            jax.ShapeDtypeStruct((8, _NB), jnp.int32),
            jax.ShapeDtypeStruct((8, _NB), jnp.int32),
            jax.ShapeDtypeStruct((E, H_PER_EXPERT, D), jnp.bfloat16),
            jax.ShapeDtypeStruct((E, H_PER_EXPERT, D), jnp.bfloat16),
            jax.ShapeDtypeStruct((E, OUTPUT_SIZE, H_PER_EXPERT), jnp.bfloat16),
        ],
    )(xf, Wg1, Wg2, W_gate, W_up, W_down)

    importance = imp[0]
    load = cnt[0]
    balance_loss = loss[0, 0]

    tok_ids = (jnp.arange(_P, dtype=jnp.int32) // NUM_SELECTS).reshape(
        _NW, _NCH, _GCH)
    pos_w = vpos.reshape(_NW, _NCH, _GCH)

    xs = _sc_dispatch(x16i, tok_ids, pos_w)

    out_slots = pl.pallas_call(
        _ffn_kernel,
        grid_spec=pltpu.PrefetchScalarGridSpec(
            num_scalar_prefetch=2,
            grid=(_NB,),
            in_specs=[
                pl.BlockSpec((_BT, _DW), lambda b, be_, vb_: (vb_[b], 0)),
                pl.BlockSpec((1, H_PER_EXPERT, D),
                             lambda b, be_, vb_: (be_[b], 0, 0)),
                pl.BlockSpec((1, H_PER_EXPERT, D),
                             lambda b, be_, vb_: (be_[b], 0, 0)),
                pl.BlockSpec((1, OUTPUT_SIZE, H_PER_EXPERT),
                             lambda b, be_, vb_: (be_[b], 0, 0)),
            ],
            out_specs=pl.BlockSpec((_BT, _OW),
                                   lambda b, be_, vb_: (vb_[b], 0)),
        ),
        out_shape=jax.ShapeDtypeStruct((_V_ROWS, _OW), jnp.int32),
    )(be[0], vb[0], xs, wg16, wu16, wd16)

    out_pairs = _sc_collect(out_slots, pos_w)

    y = pl.pallas_call(
        _cmb_kernel,
        grid=(T // _CMB_BT,),
        in_specs=[
            pl.BlockSpec((_CMB_BT, 2, _OW), lambda i: (i, 0, 0)),
            pl.BlockSpec((_CMB_BT, 2), lambda i: (i, 0)),
        ],
        out_specs=pl.BlockSpec((_CMB_BT, OUTPUT_SIZE), lambda i: (i, 0)),
        out_shape=jax.ShapeDtypeStruct((T, OUTPUT_SIZE), jnp.float32),
    )(out_pairs.reshape(T, 2, _OW), esc)

    return (y.reshape(B, S, OUTPUT_SIZE), balance_loss, load, importance)
